# Initial kernel scaffold; baseline (speedup 1.0000x reference)
import jax
import jax.numpy as jnp
from jax import lax
from jax.experimental import pallas as pl
from jax.experimental.pallas import tpu as pltpu
from jax.experimental.pallas import tpu_sc as plsc

N = 10000
D = 256
H = 128
E = 160000

NC, NS, L = 2, 16, 16
NW = NC * NS

EPAD = 163840
CHUNK = 128
NCHUNK = EPAD // NS // CHUNK
DEG_E = EPAD // NW
DEG_N = 10016
NPAD = 10400
STRIPE = NPAD // NS
BLK = 400
GRID = N // BLK

_mesh = plsc.VectorSubcoreMesh(
    core_axis_name="c", subcore_axis_name="s", num_cores=NC, num_subcores=NS)


def _deg_body(dst_hbm, out_hbm, dst_v, acc_v):
    wid = lax.axis_index("c") * NS + lax.axis_index("s")
    pltpu.sync_copy(dst_hbm.at[wid], dst_v)

    def zero(i, _):
        acc_v[pl.ds(i * L, L)] = jnp.zeros((L,), jnp.float32)
        return 0
    lax.fori_loop(0, DEG_N // L, zero, 0)

    ones = jnp.ones((L,), jnp.float32)

    def body(i, _):
        idx = dst_v[pl.ds(i * L, L)]
        plsc.addupdate_scatter(acc_v, [idx], ones)
        return 0
    lax.fori_loop(0, DEG_E // L, body, 0)
    pltpu.sync_copy(acc_v, out_hbm.at[wid])


def _degree(dst_grouped):
    f = pl.kernel(
        _deg_body,
        out_type=jax.ShapeDtypeStruct((NW, DEG_N), jnp.float32),
        mesh=_mesh,
        scratch_types=[
            pltpu.VMEM((DEG_E,), jnp.int32),
            pltpu.VMEM((DEG_N,), jnp.float32),
        ],
    )
    return f(dst_grouped)


def _agg_body(table_hbm, src_hbm, dst_hbm, zeros_hbm, out_hbm,
              src_v, dst_v, rows_v, acc_sh, sem):
    c = lax.axis_index("c")
    s = lax.axis_index("s")
    pltpu.sync_copy(src_hbm.at[s], src_v)
    pltpu.sync_copy(dst_hbm.at[s], dst_v)

    off = c * N

    def shift(i, _):
        j = i // (CHUNK // L)
        k = i % (CHUNK // L)
        src_v[j, pl.ds(k * L, L)] = src_v[j, pl.ds(k * L, L)] + off
        return 0
    lax.fori_loop(0, NCHUNK * (CHUNK // L), shift, 0)

    pltpu.sync_copy(zeros_hbm, acc_sh.at[pl.ds(s * STRIPE, STRIPE)])
    plsc.subcore_barrier()

    def chunk(j, _):
        pltpu.async_copy(table_hbm.at[src_v.at[j]], rows_v, sem).wait()
        pltpu.sync_copy(rows_v, acc_sh.at[dst_v.at[j]], add=True)
        return 0
    lax.fori_loop(0, NCHUNK, chunk, 0)

    plsc.subcore_barrier()
    pltpu.sync_copy(acc_sh.at[pl.ds(s * STRIPE, STRIPE)],
                    out_hbm.at[pl.ds(c * NPAD + s * STRIPE, STRIPE)])


def _aggregate(table, src3, dst3, zeros):
    f = pl.kernel(
        _agg_body,
        out_type=jax.ShapeDtypeStruct((NC * NPAD, H), jnp.float32),
        mesh=_mesh,
        scratch_types=[
            pltpu.VMEM((NCHUNK, CHUNK), jnp.int32),
            pltpu.VMEM((NCHUNK, CHUNK), jnp.int32),
            pltpu.VMEM((CHUNK, H), jnp.float32),
            pltpu.VMEM_SHARED((NPAD, H), jnp.float32),
            pltpu.SemaphoreType.DMA,
        ],
    )
    return f(table, src3, dst3, zeros)


def _dinv_from(deg_blk):
    return lax.rsqrt(jnp.sum(deg_blk, axis=0) + 1.0)


def _mm1_body(x_ref, w_ref, deg_ref, z_ref):
    dinv = _dinv_from(deg_ref[...])
    z = jnp.dot(x_ref[...], w_ref[...],
                preferred_element_type=jnp.float32) * dinv[:, None]
    z_ref[0] = z[:, :H]
    z_ref[1] = z[:, H:]


def _mm1(x, w, deg):
    return pl.pallas_call(
        _mm1_body,
        grid=(GRID,),
        in_specs=[
            pl.BlockSpec((BLK, D), lambda i: (i, 0)),
            pl.BlockSpec((D, D), lambda i: (0, 0)),
            pl.BlockSpec((NW, BLK), lambda i: (0, i)),
        ],
        out_specs=pl.BlockSpec((2, BLK, H), lambda i: (0, i, 0)),
        out_shape=jax.ShapeDtypeStruct((2, N, H), jnp.float32),
    )(x, w, deg)


def _combine_mm_body(s0_ref, s1_ref, z_ref, deg_ref, b_ref, w_ref,
                     x1_ref, z2_ref):
    dinv = _dinv_from(deg_ref[...])
    agg = jnp.concatenate([s0_ref[...], s1_ref[...]], axis=1)
    zl = jnp.concatenate([z_ref[0], z_ref[1]], axis=1)
    x1 = jax.nn.relu((agg + zl) * dinv[:, None] + b_ref[...])
    x1_ref[...] = x1
    z2 = jnp.dot(x1, w_ref[...],
                 preferred_element_type=jnp.float32) * dinv[:, None]
    z2_ref[0] = z2[:, :H]
    z2_ref[1] = z2[:, H:]


def _combine_mm(s_flat, z, deg, b, w):
    return pl.pallas_call(
        _combine_mm_body,
        grid=(GRID,),
        in_specs=[
            pl.BlockSpec((BLK, H), lambda i: (i, 0)),
            pl.BlockSpec((BLK, H), lambda i: (i + NPAD // BLK, 0)),
            pl.BlockSpec((2, BLK, H), lambda i: (0, i, 0)),
            pl.BlockSpec((NW, BLK), lambda i: (0, i)),
            pl.BlockSpec((1, D), lambda i: (0, 0)),
            pl.BlockSpec((D, D), lambda i: (0, 0)),
        ],
        out_specs=[
            pl.BlockSpec((BLK, D), lambda i: (i, 0)),
            pl.BlockSpec((2, BLK, H), lambda i: (0, i, 0)),
        ],
        out_shape=[
            jax.ShapeDtypeStruct((N, D), jnp.float32),
            jax.ShapeDtypeStruct((2, N, H), jnp.float32),
        ],
    )(s_flat, s_flat, z, deg, b, w)


def _final_body(s0_ref, s1_ref, z_ref, deg_ref, b_ref, x1_ref,
                x2_ref, y2_ref):
    dinv = _dinv_from(deg_ref[...])
    agg = jnp.concatenate([s0_ref[...], s1_ref[...]], axis=1)
    zl = jnp.concatenate([z_ref[0], z_ref[1]], axis=1)
    x2 = jax.nn.relu((agg + zl) * dinv[:, None] + b_ref[...])
    x2_ref[...] = x2
    y2_ref[...] = x2 - x1_ref[...]


def _final(s_flat, z, deg, b, x1):
    return pl.pallas_call(
        _final_body,
        grid=(GRID,),
        in_specs=[
            pl.BlockSpec((BLK, H), lambda i: (i, 0)),
            pl.BlockSpec((BLK, H), lambda i: (i + NPAD // BLK, 0)),
            pl.BlockSpec((2, BLK, H), lambda i: (0, i, 0)),
            pl.BlockSpec((NW, BLK), lambda i: (0, i)),
            pl.BlockSpec((1, D), lambda i: (0, 0)),
            pl.BlockSpec((BLK, D), lambda i: (i, 0)),
        ],
        out_specs=[
            pl.BlockSpec((BLK, D), lambda i: (i, 0)),
            pl.BlockSpec((BLK, D), lambda i: (i, 0)),
        ],
        out_shape=[
            jax.ShapeDtypeStruct((N, D), jnp.float32),
            jax.ShapeDtypeStruct((N, D), jnp.float32),
        ],
    )(s_flat, s_flat, z, deg, b, x1)


def kernel(X0, Y0, edge_index, W1, b1, W2, b2):
    del Y0
    src = edge_index[0].astype(jnp.int32)
    dst = edge_index[1].astype(jnp.int32)
    pad = EPAD - E
    src_pad = jnp.concatenate([src, jnp.zeros((pad,), jnp.int32)])
    dst_pad = jnp.concatenate([dst, jnp.full((pad,), N, jnp.int32)])
    src3 = src_pad.reshape(NS, NCHUNK, CHUNK)
    dst3 = dst_pad.reshape(NS, NCHUNK, CHUNK)
    dst_deg = dst_pad.reshape(NW, DEG_E)
    zeros = jnp.zeros((STRIPE, H), jnp.float32)
    b1r = b1.reshape(1, D)
    b2r = b2.reshape(1, D)

    deg = _degree(dst_deg)

    z1 = _mm1(X0, W1, deg)
    s1 = _aggregate(z1.reshape(NC * N, H), src3, dst3, zeros)
    x1, z2 = _combine_mm(s1, z1, deg, b1r, W2)
    s2 = _aggregate(z2.reshape(NC * N, H), src3, dst3, zeros)
    x2, y2 = _final(s2, z2, deg, b2r, x1)
    return (x2, y2)


# speedup vs baseline: 6.7403x; 6.7403x over previous
import jax
import jax.numpy as jnp
from jax import lax
from jax.experimental import pallas as pl
from jax.experimental.pallas import tpu as pltpu
from jax.experimental.pallas import tpu_sc as plsc

N = 10000
D = 256
H = 128
E = 160000

NC, NS, L = 2, 16, 16
NW = NC * NS

EPAD = 163840
CHUNK = 128
NCHUNK = EPAD // NS // CHUNK
DEG_E = EPAD // NW
NPAD = 10240
STRIPE = NPAD // NS
NSLC = NPAD // NW
BLK = 512
GRID = NPAD // BLK

_mesh = plsc.VectorSubcoreMesh(
    core_axis_name="c", subcore_axis_name="s", num_cores=NC, num_subcores=NS)
_sc_params = pltpu.CompilerParams(needs_layout_passes=False)


def _deg_body(dst_hbm, out_hbm, dst_v, acc_v):
    wid = lax.axis_index("c") * NS + lax.axis_index("s")
    pltpu.sync_copy(dst_hbm.at[wid], dst_v)

    def zero(i, _):
        acc_v[pl.ds(i * L, L)] = jnp.zeros((L,), jnp.float32)
        return 0
    lax.fori_loop(0, NPAD // L, zero, 0)

    ones = jnp.ones((L,), jnp.float32)

    def body(i, _):
        idx = dst_v[pl.ds(i * L, L)]
        plsc.addupdate_scatter(acc_v, [idx], ones)
        return 0
    lax.fori_loop(0, DEG_E // L, body, 0)
    pltpu.sync_copy(acc_v, out_hbm.at[pl.ds(wid * NPAD, NPAD)])


def _degree(dst_grouped):
    f = pl.kernel(
        _deg_body,
        out_type=jax.ShapeDtypeStruct((NW * NPAD,), jnp.float32),
        mesh=_mesh,
        compiler_params=_sc_params,
        scratch_types=[
            pltpu.VMEM((DEG_E,), jnp.int32),
            pltpu.VMEM((NPAD,), jnp.float32),
        ],
    )
    return f(dst_grouped)


def _dinv_body(part_hbm, out_hbm, buf_v, dinv_v):
    wid = lax.axis_index("c") * NS + lax.axis_index("s")

    def fetch(r, _):
        pltpu.sync_copy(part_hbm.at[pl.ds(r * NPAD + wid * NSLC, NSLC)],
                        buf_v.at[pl.ds(r * NSLC, NSLC)])
        return 0
    lax.fori_loop(0, NW, fetch, 0)

    def col(t, _):
        def red(r, a):
            return a + buf_v[pl.ds(r * NSLC + t * L, L)]
        deg = lax.fori_loop(0, NW, red, jnp.zeros((L,), jnp.float32))
        x = deg + 1.0
        i = plsc.bitcast(x, jnp.int32)
        y = plsc.bitcast(jnp.int32(0x5F3759DF) - (i >> 1), jnp.float32)
        hx = 0.5 * x
        y = y * (1.5 - hx * y * y)
        y = y * (1.5 - hx * y * y)
        y = y * (1.5 - hx * y * y)
        dinv_v[pl.ds(t * L, L)] = y
        return 0
    lax.fori_loop(0, NSLC // L, col, 0)
    pltpu.sync_copy(dinv_v, out_hbm.at[pl.ds(wid * NSLC, NSLC)])


def _dinv(partials):
    f = pl.kernel(
        _dinv_body,
        out_type=jax.ShapeDtypeStruct((NPAD,), jnp.float32),
        mesh=_mesh,
        compiler_params=_sc_params,
        scratch_types=[
            pltpu.VMEM((NW * NSLC,), jnp.float32),
            pltpu.VMEM((NSLC,), jnp.float32),
        ],
    )
    return f(partials)


def _agg_body(table_hbm, src_hbm, dst_hbm, zeros_hbm, out_hbm,
              src_v, dst_v, rows_v, acc_sh, sem):
    c = lax.axis_index("c")
    s = lax.axis_index("s")
    pltpu.sync_copy(src_hbm.at[s], src_v)
    pltpu.sync_copy(dst_hbm.at[s], dst_v)

    off = c * NPAD

    def shift(i, _):
        j = i // (CHUNK // L)
        k = i % (CHUNK // L)
        src_v[j, pl.ds(k * L, L)] = src_v[j, pl.ds(k * L, L)] + off
        return 0
    lax.fori_loop(0, NCHUNK * (CHUNK // L), shift, 0)

    pltpu.sync_copy(zeros_hbm, acc_sh.at[pl.ds(s * STRIPE, STRIPE)])
    plsc.subcore_barrier()

    def chunk(j, _):
        pltpu.async_copy(table_hbm.at[src_v.at[j]], rows_v, sem).wait()
        pltpu.sync_copy(rows_v, acc_sh.at[dst_v.at[j]], add=True)
        return 0
    lax.fori_loop(0, NCHUNK, chunk, 0)

    plsc.subcore_barrier()
    pltpu.sync_copy(acc_sh.at[pl.ds(s * STRIPE, STRIPE)],
                    out_hbm.at[pl.ds(c * NPAD + s * STRIPE, STRIPE)])


def _aggregate(table, src3, dst3, zeros):
    f = pl.kernel(
        _agg_body,
        out_type=jax.ShapeDtypeStruct((NC * NPAD, H), jnp.float32),
        mesh=_mesh,
        compiler_params=_sc_params,
        scratch_types=[
            pltpu.VMEM((NCHUNK, CHUNK), jnp.int32),
            pltpu.VMEM((NCHUNK, CHUNK), jnp.int32),
            pltpu.VMEM((CHUNK, H), jnp.float32),
            pltpu.VMEM_SHARED((NPAD, H), jnp.float32),
            pltpu.SemaphoreType.DMA,
        ],
    )
    return f(table, src3, dst3, zeros)


def _mm1_body(x_ref, w_ref, dinv_ref, z_ref):
    z = jnp.dot(x_ref[...], w_ref[...],
                preferred_element_type=jnp.float32) * dinv_ref[...]
    z_ref[0] = z[:, :H]
    z_ref[1] = z[:, H:]


def _mm1(x, w, dinv):
    return pl.pallas_call(
        _mm1_body,
        grid=(GRID,),
        in_specs=[
            pl.BlockSpec((BLK, D), lambda i: (i, 0)),
            pl.BlockSpec((D, D), lambda i: (0, 0)),
            pl.BlockSpec((BLK, 1), lambda i: (i, 0)),
        ],
        out_specs=pl.BlockSpec((2, BLK, H), lambda i: (0, i, 0)),
        out_shape=jax.ShapeDtypeStruct((2, NPAD, H), jnp.float32),
    )(x, w, dinv)


def _combine_mm_body(s0_ref, s1_ref, z_ref, dinv_ref, b_ref, w_ref,
                     x1_ref, z2_ref):
    dinv = dinv_ref[...]
    agg = jnp.concatenate([s0_ref[...], s1_ref[...]], axis=1)
    zl = jnp.concatenate([z_ref[0], z_ref[1]], axis=1)
    x1 = jax.nn.relu((agg + zl) * dinv + b_ref[...])
    x1_ref[...] = x1
    z2 = jnp.dot(x1, w_ref[...],
                 preferred_element_type=jnp.float32) * dinv
    z2_ref[0] = z2[:, :H]
    z2_ref[1] = z2[:, H:]


def _combine_mm(s_flat, z, dinv, b, w):
    return pl.pallas_call(
        _combine_mm_body,
        grid=(GRID,),
        in_specs=[
            pl.BlockSpec((BLK, H), lambda i: (i, 0)),
            pl.BlockSpec((BLK, H), lambda i: (i + GRID, 0)),
            pl.BlockSpec((2, BLK, H), lambda i: (0, i, 0)),
            pl.BlockSpec((BLK, 1), lambda i: (i, 0)),
            pl.BlockSpec((1, D), lambda i: (0, 0)),
            pl.BlockSpec((D, D), lambda i: (0, 0)),
        ],
        out_specs=[
            pl.BlockSpec((BLK, D), lambda i: (i, 0)),
            pl.BlockSpec((2, BLK, H), lambda i: (0, i, 0)),
        ],
        out_shape=[
            jax.ShapeDtypeStruct((NPAD, D), jnp.float32),
            jax.ShapeDtypeStruct((2, NPAD, H), jnp.float32),
        ],
    )(s_flat, s_flat, z, dinv, b, w)


def _final_body(s0_ref, s1_ref, z_ref, dinv_ref, b_ref, x1_ref,
                x2_ref, y2_ref):
    agg = jnp.concatenate([s0_ref[...], s1_ref[...]], axis=1)
    zl = jnp.concatenate([z_ref[0], z_ref[1]], axis=1)
    x2 = jax.nn.relu((agg + zl) * dinv_ref[...] + b_ref[...])
    x2_ref[...] = x2
    y2_ref[...] = x2 - x1_ref[...]


def _final(s_flat, z, dinv, b, x1):
    return pl.pallas_call(
        _final_body,
        grid=(GRID,),
        in_specs=[
            pl.BlockSpec((BLK, H), lambda i: (i, 0)),
            pl.BlockSpec((BLK, H), lambda i: (i + GRID, 0)),
            pl.BlockSpec((2, BLK, H), lambda i: (0, i, 0)),
            pl.BlockSpec((BLK, 1), lambda i: (i, 0)),
            pl.BlockSpec((1, D), lambda i: (0, 0)),
            pl.BlockSpec((BLK, D), lambda i: (i, 0)),
        ],
        out_specs=[
            pl.BlockSpec((BLK, D), lambda i: (i, 0)),
            pl.BlockSpec((BLK, D), lambda i: (i, 0)),
        ],
        out_shape=[
            jax.ShapeDtypeStruct((NPAD, D), jnp.float32),
            jax.ShapeDtypeStruct((NPAD, D), jnp.float32),
        ],
    )(s_flat, s_flat, z, dinv, b, x1)


def kernel(X0, Y0, edge_index, W1, b1, W2, b2):
    del Y0
    src = edge_index[0].astype(jnp.int32)
    dst = edge_index[1].astype(jnp.int32)
    pad = EPAD - E
    src_pad = jnp.concatenate([src, jnp.zeros((pad,), jnp.int32)])
    dst_pad = jnp.concatenate([dst, jnp.full((pad,), N, jnp.int32)])
    src3 = src_pad.reshape(NS, NCHUNK, CHUNK)
    dst3 = dst_pad.reshape(NS, NCHUNK, CHUNK)
    dst_deg = dst_pad.reshape(NW, DEG_E)
    zeros = jnp.zeros((STRIPE, H), jnp.float32)
    x0p = jnp.pad(X0, ((0, NPAD - N), (0, 0)))
    b1r = b1.reshape(1, D)
    b2r = b2.reshape(1, D)

    parts = _degree(dst_deg)
    dinv = _dinv(parts).reshape(NPAD, 1)

    z1 = _mm1(x0p, W1, dinv)
    s1 = _aggregate(z1.reshape(NC * NPAD, H), src3, dst3, zeros)
    x1, z2 = _combine_mm(s1, z1, dinv, b1r, W2)
    s2 = _aggregate(z2.reshape(NC * NPAD, H), src3, dst3, zeros)
    x2, y2 = _final(s2, z2, dinv, b2r, x1)
    return (x2[:N], y2[:N])


# speedup vs baseline: 8.3451x; 1.2381x over previous
import jax
import jax.numpy as jnp
from jax import lax
from jax.experimental import pallas as pl
from jax.experimental.pallas import tpu as pltpu
from jax.experimental.pallas import tpu_sc as plsc

N = 10000
D = 256
H = 128
E = 160000

NC, NS, L = 2, 16, 16
NW = NC * NS

EPAD = 163840
CHUNK = 128
NCHUNK = EPAD // NS // CHUNK
DEG_E = EPAD // NW
NPAD = 10240
STRIPE = NPAD // NS
NSLC = NPAD // NW
BLK = 512
GRID = NPAD // BLK

_mesh = plsc.VectorSubcoreMesh(
    core_axis_name="c", subcore_axis_name="s", num_cores=NC, num_subcores=NS)
_sc_params = pltpu.CompilerParams(needs_layout_passes=False)


def _deg_body(dst_hbm, out_hbm, dst_v, acc_v):
    wid = lax.axis_index("c") * NS + lax.axis_index("s")
    pltpu.sync_copy(dst_hbm.at[wid], dst_v)

    def zero(i, _):
        acc_v[pl.ds(i * L, L)] = jnp.zeros((L,), jnp.float32)
        return 0
    lax.fori_loop(0, NPAD // L, zero, 0)

    ones = jnp.ones((L,), jnp.float32)

    def body(i, _):
        idx = dst_v[pl.ds(i * L, L)]
        plsc.addupdate_scatter(acc_v, [idx], ones)
        return 0
    lax.fori_loop(0, DEG_E // L, body, 0)
    pltpu.sync_copy(acc_v, out_hbm.at[pl.ds(wid * NPAD, NPAD)])


def _degree(dst_grouped):
    f = pl.kernel(
        _deg_body,
        out_type=jax.ShapeDtypeStruct((NW * NPAD,), jnp.float32),
        mesh=_mesh,
        compiler_params=_sc_params,
        scratch_types=[
            pltpu.VMEM((DEG_E,), jnp.int32),
            pltpu.VMEM((NPAD,), jnp.float32),
        ],
    )
    return f(dst_grouped)


def _dinv_body(part_hbm, out_hbm, buf_v, dinv_v):
    wid = lax.axis_index("c") * NS + lax.axis_index("s")

    def fetch(r, _):
        pltpu.sync_copy(part_hbm.at[pl.ds(r * NPAD + wid * NSLC, NSLC)],
                        buf_v.at[pl.ds(r * NSLC, NSLC)])
        return 0
    lax.fori_loop(0, NW, fetch, 0)

    def col(t, _):
        def red(r, a):
            return a + buf_v[pl.ds(r * NSLC + t * L, L)]
        deg = lax.fori_loop(0, NW, red, jnp.zeros((L,), jnp.float32))
        x = deg + 1.0
        i = plsc.bitcast(x, jnp.int32)
        y = plsc.bitcast(jnp.int32(0x5F3759DF) - (i >> 1), jnp.float32)
        hx = 0.5 * x
        y = y * (1.5 - hx * y * y)
        y = y * (1.5 - hx * y * y)
        y = y * (1.5 - hx * y * y)
        dinv_v[pl.ds(t * L, L)] = y
        return 0
    lax.fori_loop(0, NSLC // L, col, 0)
    pltpu.sync_copy(dinv_v, out_hbm.at[pl.ds(wid * NSLC, NSLC)])


def _dinv(partials):
    f = pl.kernel(
        _dinv_body,
        out_type=jax.ShapeDtypeStruct((NPAD,), jnp.float32),
        mesh=_mesh,
        compiler_params=_sc_params,
        scratch_types=[
            pltpu.VMEM((NW * NSLC,), jnp.float32),
            pltpu.VMEM((NSLC,), jnp.float32),
        ],
    )
    return f(partials)


NBUF = 2


def _agg_body(table_hbm, src_hbm, dst_hbm, zeros_hbm, out_hbm,
              src_v, dst0, dst1, rows0, rows1,
              gsem0, gsem1, dsem0, dsem1, acc_sh):
    c = lax.axis_index("c")
    s = lax.axis_index("s")
    pltpu.sync_copy(src_hbm.at[c * NS + s], src_v)

    pltpu.sync_copy(zeros_hbm, acc_sh.at[pl.ds(s * STRIPE, STRIPE)])
    plsc.subcore_barrier()

    rows = (rows0, rows1)
    dstb = (dst0, dst1)
    gsem = (gsem0, gsem1)
    dsem = (dsem0, dsem1)
    drow = s * NCHUNK

    for b in range(NBUF):
        pltpu.async_copy(table_hbm.at[src_v.at[b]], rows[b], gsem[b])
        pltpu.async_copy(dst_hbm.at[drow + b], dstb[b].at[0], dsem[b])

    def step(i, _):
        for b in range(NBUF):
            j = i * NBUF + b
            pltpu.make_async_copy(
                table_hbm.at[src_v.at[j]], rows[b], gsem[b]).wait()
            pltpu.make_async_copy(
                dst_hbm.at[drow + j], dstb[b].at[0], dsem[b]).wait()
            pltpu.sync_copy(rows[b], acc_sh.at[dstb[b].at[0]], add=True)

            @pl.when(j < NCHUNK - NBUF)
            def _():
                pltpu.async_copy(
                    table_hbm.at[src_v.at[j + NBUF]], rows[b], gsem[b])
                pltpu.async_copy(
                    dst_hbm.at[drow + j + NBUF], dstb[b].at[0], dsem[b])
        return 0
    lax.fori_loop(0, NCHUNK // NBUF, step, 0)

    plsc.subcore_barrier()
    pltpu.sync_copy(acc_sh.at[pl.ds(s * STRIPE, STRIPE)],
                    out_hbm.at[pl.ds(c * NPAD + s * STRIPE, STRIPE)])


def _aggregate(table, src4, dst3, zeros):
    f = pl.kernel(
        _agg_body,
        out_type=jax.ShapeDtypeStruct((NC * NPAD, H), jnp.float32),
        mesh=_mesh,
        compiler_params=_sc_params,
        scratch_types=[
            pltpu.VMEM((NCHUNK, CHUNK), jnp.int32),
            pltpu.VMEM((1, CHUNK), jnp.int32),
            pltpu.VMEM((1, CHUNK), jnp.int32),
            pltpu.VMEM((CHUNK, H), jnp.float32),
            pltpu.VMEM((CHUNK, H), jnp.float32),
            pltpu.SemaphoreType.DMA,
            pltpu.SemaphoreType.DMA,
            pltpu.SemaphoreType.DMA,
            pltpu.SemaphoreType.DMA,
            pltpu.VMEM_SHARED((NPAD, H), jnp.float32),
        ],
    )
    return f(table, src4, dst3, zeros)


def _mm1_body(x_ref, w_ref, dinv_ref, z_ref):
    z = jnp.dot(x_ref[...], w_ref[...],
                preferred_element_type=jnp.float32) * dinv_ref[...]
    z_ref[0] = z[:, :H]
    z_ref[1] = z[:, H:]


def _mm1(x, w, dinv):
    return pl.pallas_call(
        _mm1_body,
        grid=(GRID,),
        in_specs=[
            pl.BlockSpec((BLK, D), lambda i: (i, 0)),
            pl.BlockSpec((D, D), lambda i: (0, 0)),
            pl.BlockSpec((BLK, 1), lambda i: (i, 0)),
        ],
        out_specs=pl.BlockSpec((2, BLK, H), lambda i: (0, i, 0)),
        out_shape=jax.ShapeDtypeStruct((2, NPAD, H), jnp.float32),
    )(x, w, dinv)


def _combine_mm_body(s0_ref, s1_ref, z_ref, dinv_ref, b_ref, w_ref,
                     x1_ref, z2_ref):
    dinv = dinv_ref[...]
    agg = jnp.concatenate([s0_ref[...], s1_ref[...]], axis=1)
    zl = jnp.concatenate([z_ref[0], z_ref[1]], axis=1)
    x1 = jax.nn.relu((agg + zl) * dinv + b_ref[...])
    x1_ref[...] = x1
    z2 = jnp.dot(x1, w_ref[...],
                 preferred_element_type=jnp.float32) * dinv
    z2_ref[0] = z2[:, :H]
    z2_ref[1] = z2[:, H:]


def _combine_mm(s_flat, z, dinv, b, w):
    return pl.pallas_call(
        _combine_mm_body,
        grid=(GRID,),
        in_specs=[
            pl.BlockSpec((BLK, H), lambda i: (i, 0)),
            pl.BlockSpec((BLK, H), lambda i: (i + GRID, 0)),
            pl.BlockSpec((2, BLK, H), lambda i: (0, i, 0)),
            pl.BlockSpec((BLK, 1), lambda i: (i, 0)),
            pl.BlockSpec((1, D), lambda i: (0, 0)),
            pl.BlockSpec((D, D), lambda i: (0, 0)),
        ],
        out_specs=[
            pl.BlockSpec((BLK, D), lambda i: (i, 0)),
            pl.BlockSpec((2, BLK, H), lambda i: (0, i, 0)),
        ],
        out_shape=[
            jax.ShapeDtypeStruct((NPAD, D), jnp.float32),
            jax.ShapeDtypeStruct((2, NPAD, H), jnp.float32),
        ],
    )(s_flat, s_flat, z, dinv, b, w)


def _final_body(s0_ref, s1_ref, z_ref, dinv_ref, b_ref, x1_ref,
                x2_ref, y2_ref):
    agg = jnp.concatenate([s0_ref[...], s1_ref[...]], axis=1)
    zl = jnp.concatenate([z_ref[0], z_ref[1]], axis=1)
    x2 = jax.nn.relu((agg + zl) * dinv_ref[...] + b_ref[...])
    x2_ref[...] = x2
    y2_ref[...] = x2 - x1_ref[...]


def _final(s_flat, z, dinv, b, x1):
    return pl.pallas_call(
        _final_body,
        grid=(GRID,),
        in_specs=[
            pl.BlockSpec((BLK, H), lambda i: (i, 0)),
            pl.BlockSpec((BLK, H), lambda i: (i + GRID, 0)),
            pl.BlockSpec((2, BLK, H), lambda i: (0, i, 0)),
            pl.BlockSpec((BLK, 1), lambda i: (i, 0)),
            pl.BlockSpec((1, D), lambda i: (0, 0)),
            pl.BlockSpec((BLK, D), lambda i: (i, 0)),
        ],
        out_specs=[
            pl.BlockSpec((BLK, D), lambda i: (i, 0)),
            pl.BlockSpec((BLK, D), lambda i: (i, 0)),
        ],
        out_shape=[
            jax.ShapeDtypeStruct((NPAD, D), jnp.float32),
            jax.ShapeDtypeStruct((NPAD, D), jnp.float32),
        ],
    )(s_flat, s_flat, z, dinv, b, x1)


def kernel(X0, Y0, edge_index, W1, b1, W2, b2):
    del Y0
    src = edge_index[0].astype(jnp.int32)
    dst = edge_index[1].astype(jnp.int32)
    pad = EPAD - E
    src_pad = jnp.concatenate([src, jnp.zeros((pad,), jnp.int32)])
    dst_pad = jnp.concatenate([dst, jnp.full((pad,), N, jnp.int32)])
    src3 = src_pad.reshape(NS, NCHUNK, CHUNK)
    src4 = jnp.concatenate([src3, src3 + NPAD]).reshape(NW, NCHUNK, CHUNK)
    dst3 = dst_pad.reshape(NS * NCHUNK, CHUNK)
    dst_deg = dst_pad.reshape(NW, DEG_E)
    zeros = jnp.zeros((STRIPE, H), jnp.float32)
    x0p = jnp.pad(X0, ((0, NPAD - N), (0, 0)))
    b1r = b1.reshape(1, D)
    b2r = b2.reshape(1, D)

    parts = _degree(dst_deg)
    dinv = _dinv(parts).reshape(NPAD, 1)

    z1 = _mm1(x0p, W1, dinv)
    s1 = _aggregate(z1.reshape(NC * NPAD, H), src4, dst3, zeros)
    x1, z2 = _combine_mm(s1, z1, dinv, b1r, W2)
    s2 = _aggregate(z2.reshape(NC * NPAD, H), src4, dst3, zeros)
    x2, y2 = _final(s2, z2, dinv, b2r, x1)
    return (x2[:N], y2[:N])
